# Initial kernel scaffold; baseline (speedup 1.0000x reference)
import jax
import jax.numpy as jnp
from jax.experimental import pallas as pl


def kernel(x, labels, scores):
    raise NotImplementedError("write your pallas kernel here")


# speedup vs baseline: 5.0733x; 5.0733x over previous
import functools

import jax
import jax.numpy as jnp
from jax import lax
from jax.experimental import pallas as pl
from jax.experimental.pallas import tpu as pltpu
from jax.experimental.pallas import tpu_sc as plsc

_N = 1000
_C = 81
_M = 28
_D = _M * _M
_NP = 1024
_THRESH = 0.5
_F32 = jnp.float32
_HI = jax.lax.Precision.HIGHEST


@functools.lru_cache(maxsize=None)
def _make_sc_gather():
    info = plsc.get_sparse_core_info()
    nc, ns = info.num_cores, info.num_subcores
    nw = nc * ns
    b_per_w = _NP // nw
    mesh = plsc.VectorSubcoreMesh(core_axis_name="c", subcore_axis_name="s")

    @functools.partial(
        pl.kernel,
        mesh=mesh,
        out_type=jax.ShapeDtypeStruct((_NP, _D), _F32),
        scratch_types=[
            pltpu.VMEM((b_per_w,), jnp.int32),
            pltpu.VMEM((b_per_w, _D), _F32),
            pltpu.SemaphoreType.DMA,
        ],
        compiler_params=pltpu.CompilerParams(use_tc_tiling_on_sc=False),
    )
    def gather_k(table_hbm, idx_hbm, out_hbm, idx_v, rows_v, sem):
        wid = lax.axis_index("s") * nc + lax.axis_index("c")
        base = wid * b_per_w
        pltpu.sync_copy(idx_hbm.at[pl.ds(base, b_per_w)], idx_v)
        pltpu.async_copy(table_hbm.at[idx_v], rows_v, sem).wait()
        pltpu.sync_copy(rows_v, out_hbm.at[pl.ds(base, b_per_w)])

    return gather_k


def _tc_body(g_ref, srow_ref, scol_ref, out_ref, s_scr):
    masks_un = jax.nn.sigmoid(g_ref[...])
    srow = srow_ref[...]
    scol = scol_ref[...]
    ia = lax.broadcasted_iota(jnp.int32, (_NP, _NP), 0)
    ij = lax.broadcasted_iota(jnp.int32, (_NP, _NP), 1)

    beats = (scol > srow) | ((scol == srow) & (ia < ij))
    rank_row = jnp.sum(beats.astype(_F32), axis=0, keepdims=True)

    rank_i = rank_row.astype(jnp.int32)
    perm = (jnp.broadcast_to(rank_i, (_NP, _NP)) == ia).astype(_F32)
    masks_s = lax.dot_general(
        perm, masks_un, (((1,), (0,)), ((), ())),
        precision=_HI, preferred_element_type=_F32)

    areas = jnp.sum(masks_s, axis=1, keepdims=True)
    inter = lax.dot_general(
        masks_s, masks_s, (((1,), (1,)), ((), ())),
        precision=_HI, preferred_element_type=_F32)
    iou = inter / (areas + 0.0001)
    s_scr[...] = ((iou >= _THRESH) & (ij > ia)).astype(_F32)

    def body(carry):
        keep, _ = carry
        sup = lax.dot_general(
            keep, s_scr[...], (((1,), (0,)), ((), ())),
            preferred_element_type=_F32)
        new = jnp.where(sup > 0.0, 0.0, 1.0)
        changed = jnp.sum(jnp.abs(new - keep)) > 0.0
        return new, changed

    keep0 = jnp.ones((1, _NP), _F32)
    keep, _ = lax.while_loop(lambda c: c[1], body, (keep0, jnp.bool_(True)))

    eye = (ia == ij).astype(_F32)
    keep_col = jnp.sum(jnp.broadcast_to(keep, (_NP, _NP)) * eye,
                       axis=1, keepdims=True)
    out_ref[...] = masks_s * keep_col


_tc_call = pl.pallas_call(
    _tc_body,
    out_shape=jax.ShapeDtypeStruct((_NP, _D), _F32),
    scratch_shapes=[pltpu.VMEM((_NP, _NP), _F32)],
)


def kernel(x, labels, scores):
    x_flat = x.reshape(_N * _C, _D)
    gidx = jnp.arange(_N, dtype=jnp.int32) * _C + labels.astype(jnp.int32)
    gidx = jnp.concatenate([gidx, jnp.zeros((_NP - _N,), jnp.int32)])
    gathered = _make_sc_gather()(x_flat, gidx)
    sp = jnp.concatenate(
        [scores.astype(_F32), jnp.full((_NP - _N,), -1.0, _F32)])
    out = _tc_call(gathered, sp.reshape(1, _NP), sp.reshape(_NP, 1))
    return out[:_N].reshape(_N, _M, _M)


# speedup vs baseline: 5.4107x; 1.0665x over previous
import functools

import jax
import jax.numpy as jnp
from jax import lax
from jax.experimental import pallas as pl
from jax.experimental.pallas import tpu as pltpu
from jax.experimental.pallas import tpu_sc as plsc

_N = 1000
_C = 81
_M = 28
_D = _M * _M
_NP = 1024
_DP = 896
_THRESH = 0.5
_F32 = jnp.float32
_HI = jax.lax.Precision.HIGHEST


@functools.lru_cache(maxsize=None)
def _make_sc_gather():
    info = plsc.get_sparse_core_info()
    nc, ns = info.num_cores, info.num_subcores
    nw = nc * ns
    b_per_w = _NP // nw
    mesh = plsc.VectorSubcoreMesh(core_axis_name="c", subcore_axis_name="s")

    @functools.partial(
        pl.kernel,
        mesh=mesh,
        out_type=jax.ShapeDtypeStruct((_NP, _DP), _F32),
        scratch_types=[
            pltpu.VMEM((b_per_w,), jnp.int32),
            pltpu.VMEM((b_per_w, _DP), _F32),
            pltpu.SemaphoreType.DMA,
        ],
    )
    def gather_k(table_hbm, idx_hbm, out_hbm, idx_v, rows_v, sem):
        wid = lax.axis_index("s") * nc + lax.axis_index("c")
        base = wid * b_per_w
        pltpu.sync_copy(idx_hbm.at[pl.ds(base, b_per_w)], idx_v)
        pltpu.async_copy(table_hbm.at[idx_v], rows_v, sem).wait()
        pltpu.sync_copy(rows_v, out_hbm.at[pl.ds(base, b_per_w)])

    return gather_k


def _tc_body(g_ref, srow_ref, scol_ref, out_ref, s_scr):
    masks_un = jax.nn.sigmoid(g_ref[...])
    srow = srow_ref[...]
    scol = scol_ref[...]
    ia = lax.broadcasted_iota(jnp.int32, (_NP, _NP), 0)
    ij = lax.broadcasted_iota(jnp.int32, (_NP, _NP), 1)

    beats = (scol > srow) | ((scol == srow) & (ia < ij))
    rank_row = jnp.sum(beats.astype(_F32), axis=0, keepdims=True)

    rank_i = rank_row.astype(jnp.int32)
    perm = (jnp.broadcast_to(rank_i, (_NP, _NP)) == ia).astype(_F32)
    masks_s = lax.dot_general(
        perm, masks_un, (((1,), (0,)), ((), ())),
        precision=_HI, preferred_element_type=_F32)

    areas = jnp.sum(masks_s, axis=1, keepdims=True)
    inter = lax.dot_general(
        masks_s, masks_s, (((1,), (1,)), ((), ())),
        precision=_HI, preferred_element_type=_F32)
    iou = inter / (areas + 0.0001)
    s_scr[...] = ((iou >= _THRESH) & (ij > ia)).astype(_F32)

    def body(carry):
        keep, _ = carry
        sup = lax.dot_general(
            keep, s_scr[...], (((1,), (0,)), ((), ())),
            preferred_element_type=_F32)
        new = jnp.where(sup > 0.0, 0.0, 1.0)
        changed = jnp.sum(jnp.abs(new - keep)) > 0.0
        return new, changed

    keep0 = jnp.ones((1, _NP), _F32)
    keep, _ = lax.while_loop(lambda c: c[1], body, (keep0, jnp.bool_(True)))

    eye = (ia == ij).astype(_F32)
    keep_col = jnp.sum(jnp.broadcast_to(keep, (_NP, _NP)) * eye,
                       axis=1, keepdims=True)
    out_ref[...] = masks_s * keep_col


_tc_call = pl.pallas_call(
    _tc_body,
    out_shape=jax.ShapeDtypeStruct((_NP, _D), _F32),
    scratch_shapes=[pltpu.VMEM((_NP, _NP), _F32)],
)


def kernel(x, labels, scores):
    gidx = jnp.arange(_N, dtype=jnp.int32) * _C + labels.astype(jnp.int32)
    gidx = jnp.concatenate(
        [gidx, jnp.zeros((_NP - _N,), jnp.int32)])
    table = jnp.pad(x.reshape(_N * _C, _D), ((0, 0), (0, _DP - _D)))
    gathered = _make_sc_gather()(table, gidx)
    sp = jnp.concatenate(
        [scores.astype(_F32), jnp.full((_NP - _N,), -1.0, _F32)])
    out = _tc_call(gathered[:, :_D], sp.reshape(1, _NP), sp.reshape(_NP, 1))
    return out[:_N].reshape(_N, _M, _M)


# speedup vs baseline: 8.2671x; 1.5279x over previous
import functools

import jax
import jax.numpy as jnp
from jax import lax
from jax.experimental import pallas as pl
from jax.experimental.pallas import tpu as pltpu
from jax.experimental.pallas import tpu_sc as plsc

_N = 1000
_C = 81
_M = 28
_D = _M * _M
_NP = 1024
_DP = 896
_THRESH = 0.5
_F32 = jnp.float32
_HI = jax.lax.Precision.HIGHEST


@functools.lru_cache(maxsize=None)
def _make_sc_gather():
    info = plsc.get_sparse_core_info()
    nc, ns = info.num_cores, info.num_subcores
    nw = nc * ns
    b_per_w = _NP // nw
    mesh = plsc.VectorSubcoreMesh(core_axis_name="c", subcore_axis_name="s")

    @functools.partial(
        pl.kernel,
        mesh=mesh,
        out_type=jax.ShapeDtypeStruct((_NP, _M, _M), _F32),
        scratch_types=[
            pltpu.VMEM((b_per_w,), jnp.int32),
            pltpu.SemaphoreType.DMA,
        ],
    )
    def gather_k(x3_hbm, idx_hbm, out_hbm, idx_v, sem):
        wid = lax.axis_index("s") * nc + lax.axis_index("c")
        base = wid * b_per_w
        pltpu.sync_copy(idx_hbm.at[pl.ds(base, b_per_w)], idx_v)
        copies = []
        for k in range(b_per_w):
            f = idx_v[pl.ds(k, 1)][0]
            copies.append(
                pltpu.async_copy(x3_hbm.at[f], out_hbm.at[base + k], sem))
        for c in copies:
            c.wait()

    return gather_k


def _tc_body(g_ref, srow_ref, scol_ref, out_ref, s_scr):
    masks_un = jax.nn.sigmoid(g_ref[...])
    srow = srow_ref[...]
    scol = scol_ref[...]
    ia = lax.broadcasted_iota(jnp.int32, (_NP, _NP), 0)
    ij = lax.broadcasted_iota(jnp.int32, (_NP, _NP), 1)

    beats = (scol > srow) | ((scol == srow) & (ia < ij))
    rank_row = jnp.sum(beats.astype(_F32), axis=0, keepdims=True)

    rank_i = rank_row.astype(jnp.int32)
    perm = (jnp.broadcast_to(rank_i, (_NP, _NP)) == ia).astype(_F32)
    masks_s = lax.dot_general(
        perm, masks_un, (((1,), (0,)), ((), ())),
        precision=_HI, preferred_element_type=_F32)

    areas = jnp.sum(masks_s, axis=1, keepdims=True)
    inter = lax.dot_general(
        masks_s, masks_s, (((1,), (1,)), ((), ())),
        precision=_HI, preferred_element_type=_F32)
    iou = inter / (areas + 0.0001)
    s_scr[...] = ((iou >= _THRESH) & (ij > ia)).astype(_F32)

    def body(carry):
        keep, _ = carry
        sup = lax.dot_general(
            keep, s_scr[...], (((1,), (0,)), ((), ())),
            preferred_element_type=_F32)
        new = jnp.where(sup > 0.0, 0.0, 1.0)
        changed = jnp.sum(jnp.abs(new - keep)) > 0.0
        return new, changed

    keep0 = jnp.ones((1, _NP), _F32)
    keep, _ = lax.while_loop(lambda c: c[1], body, (keep0, jnp.bool_(True)))

    eye = (ia == ij).astype(_F32)
    keep_col = jnp.sum(jnp.broadcast_to(keep, (_NP, _NP)) * eye,
                       axis=1, keepdims=True)
    out_ref[...] = masks_s * keep_col


_tc_call = pl.pallas_call(
    _tc_body,
    out_shape=jax.ShapeDtypeStruct((_NP, _D), _F32),
    scratch_shapes=[pltpu.VMEM((_NP, _NP), _F32)],
)


def kernel(x, labels, scores):
    gidx = jnp.arange(_N, dtype=jnp.int32) * _C + labels.astype(jnp.int32)
    gidx = jnp.concatenate(
        [gidx, jnp.zeros((_NP - _N,), jnp.int32)])
    gathered = _make_sc_gather()(x.reshape(_N * _C, _M, _M), gidx)
    sp = jnp.concatenate(
        [scores.astype(_F32), jnp.full((_NP - _N,), -1.0, _F32)])
    out = _tc_call(gathered.reshape(_NP, _D),
                   sp.reshape(1, _NP), sp.reshape(_NP, 1))
    return out[:_N].reshape(_N, _M, _M)


# speedup vs baseline: 8.3392x; 1.0087x over previous
import functools

import jax
import jax.numpy as jnp
from jax import lax
from jax.experimental import pallas as pl
from jax.experimental.pallas import tpu as pltpu
from jax.experimental.pallas import tpu_sc as plsc

_N = 1000
_C = 81
_M = 28
_D = _M * _M
_NP = 1024
_DP = 896
_THRESH = 0.5
_F32 = jnp.float32
_HI = jax.lax.Precision.HIGHEST


@functools.lru_cache(maxsize=None)
def _make_sc_gather():
    info = plsc.get_sparse_core_info()
    nc, ns = info.num_cores, info.num_subcores
    nw = nc * ns
    b_per_w = _NP // nw
    mesh = plsc.VectorSubcoreMesh(core_axis_name="c", subcore_axis_name="s")

    @functools.partial(
        pl.kernel,
        mesh=mesh,
        out_type=jax.ShapeDtypeStruct((_NP, _M, _M), _F32),
        scratch_types=[
            pltpu.VMEM((b_per_w,), jnp.int32),
            pltpu.SemaphoreType.DMA,
        ],
    )
    def gather_k(x4_hbm, idx_hbm, out_hbm, idx_v, sem):
        wid = lax.axis_index("s") * nc + lax.axis_index("c")
        base = wid * b_per_w
        pltpu.sync_copy(idx_hbm.at[pl.ds(base, b_per_w)], idx_v)
        copies = []
        for k in range(b_per_w):
            f = idx_v[pl.ds(k, 1)][0]
            d = f // _C
            c = f - d * _C
            copies.append(
                pltpu.async_copy(x4_hbm.at[d, c], out_hbm.at[base + k], sem))
        for c in copies:
            c.wait()

    return gather_k


def _tc_body(g_ref, srow_ref, scol_ref, out_ref, s_scr):
    masks_un = jax.nn.sigmoid(g_ref[...])
    srow = srow_ref[...]
    scol = scol_ref[...]
    ia = lax.broadcasted_iota(jnp.int32, (_NP, _NP), 0)
    ij = lax.broadcasted_iota(jnp.int32, (_NP, _NP), 1)

    beats = (scol > srow) | ((scol == srow) & (ia < ij))
    rank_row = jnp.sum(beats.astype(_F32), axis=0, keepdims=True)

    rank_i = rank_row.astype(jnp.int32)
    perm = (jnp.broadcast_to(rank_i, (_NP, _NP)) == ia).astype(_F32)
    masks_s = lax.dot_general(
        perm, masks_un, (((1,), (0,)), ((), ())),
        precision=_HI, preferred_element_type=_F32)

    areas = jnp.sum(masks_s, axis=1, keepdims=True)
    inter = lax.dot_general(
        masks_s, masks_s, (((1,), (1,)), ((), ())),
        precision=_HI, preferred_element_type=_F32)
    iou = inter / (areas + 0.0001)
    s_scr[...] = ((iou >= _THRESH) & (ij > ia)).astype(_F32)

    def body(carry):
        keep, _ = carry
        sup = lax.dot_general(
            keep, s_scr[...], (((1,), (0,)), ((), ())),
            preferred_element_type=_F32)
        new = jnp.where(sup > 0.0, 0.0, 1.0)
        changed = jnp.sum(jnp.abs(new - keep)) > 0.0
        return new, changed

    keep0 = jnp.ones((1, _NP), _F32)
    keep, _ = lax.while_loop(lambda c: c[1], body, (keep0, jnp.bool_(True)))

    eye = (ia == ij).astype(_F32)
    keep_col = jnp.sum(jnp.broadcast_to(keep, (_NP, _NP)) * eye,
                       axis=1, keepdims=True)
    out_ref[...] = masks_s * keep_col


_tc_call = pl.pallas_call(
    _tc_body,
    out_shape=jax.ShapeDtypeStruct((_NP, _D), _F32),
    scratch_shapes=[pltpu.VMEM((_NP, _NP), _F32)],
)


def kernel(x, labels, scores):
    gidx = jnp.arange(_N, dtype=jnp.int32) * _C + labels.astype(jnp.int32)
    gidx = jnp.concatenate(
        [gidx, jnp.zeros((_NP - _N,), jnp.int32)])
    gathered = _make_sc_gather()(x, gidx)
    sp = jnp.concatenate(
        [scores.astype(_F32), jnp.full((_NP - _N,), -1.0, _F32)])
    out = _tc_call(gathered.reshape(_NP, _D),
                   sp.reshape(1, _NP), sp.reshape(_NP, 1))
    return out[:_N].reshape(_N, _M, _M)


# speedup vs baseline: 80.4235x; 9.6440x over previous
import jax
import jax.numpy as jnp
from jax import lax
from jax.experimental import pallas as pl
from jax.experimental.pallas import tpu as pltpu

_N = 1000
_C = 81
_M = 28
_D = _M * _M
_NP = 1024
_BP = 16
_THRESH = 0.5
_F32 = jnp.float32
_HI = jax.lax.Precision.HIGHEST


def _gather_body(lab_ref, x_ref, out_ref):
    lab = lab_ref[...]
    acc = x_ref[:, 0, :]
    for c in range(1, _C):
        acc = jnp.where(lab == c, x_ref[:, c, :], acc)
    out_ref[...] = acc


_gather_call = pl.pallas_call(
    _gather_body,
    grid=(_D // _BP,),
    in_specs=[
        pl.BlockSpec((1, _N), lambda i: (0, 0)),
        pl.BlockSpec((_BP, _C, _N), lambda i: (i, 0, 0)),
    ],
    out_specs=pl.BlockSpec((_BP, _N), lambda i: (i, 0)),
    out_shape=jax.ShapeDtypeStruct((_D, _N), _F32),
)


def _nms_body(g_ref, srow_ref, scol_ref, out_ref, s_scr):
    masks_t = jax.nn.sigmoid(g_ref[...])
    srow = srow_ref[...]
    scol = scol_ref[...]
    ia = lax.broadcasted_iota(jnp.int32, (_NP, _NP), 0)
    ij = lax.broadcasted_iota(jnp.int32, (_NP, _NP), 1)

    beats = (scol > srow) | ((scol == srow) & (ia < ij))
    rank_row = jnp.sum(beats.astype(_F32), axis=0, keepdims=True)

    rank_i = rank_row.astype(jnp.int32)
    perm = (jnp.broadcast_to(rank_i, (_NP, _NP)) == ia).astype(_F32)
    masks_s = lax.dot_general(
        masks_t, perm, (((1,), (1,)), ((), ())),
        precision=_HI, preferred_element_type=_F32)

    ones_col = jnp.ones((_D, 1), _F32)
    areas = lax.dot_general(
        masks_s, ones_col, (((0,), (0,)), ((), ())),
        precision=_HI, preferred_element_type=_F32)
    inter = lax.dot_general(
        masks_s, masks_s, (((0,), (0,)), ((), ())),
        precision=_HI, preferred_element_type=_F32)
    iou = inter / (areas + 0.0001)
    s_scr[...] = ((iou >= _THRESH) & (ij > ia)).astype(_F32)

    def body(carry):
        keep, _ = carry
        sup = lax.dot_general(
            keep, s_scr[...], (((1,), (0,)), ((), ())),
            preferred_element_type=_F32)
        new = jnp.where(sup > 0.0, 0.0, 1.0)
        changed = jnp.sum(jnp.abs(new - keep)) > 0.0
        return new, changed

    keep0 = jnp.ones((1, _NP), _F32)
    keep, _ = lax.while_loop(lambda c: c[1], body, (keep0, jnp.bool_(True)))

    out_ref[...] = masks_s * keep


_nms_call = pl.pallas_call(
    _nms_body,
    out_shape=jax.ShapeDtypeStruct((_D, _NP), _F32),
    scratch_shapes=[pltpu.VMEM((_NP, _NP), _F32)],
)


def kernel(x, labels, scores):
    x_t = x.transpose(2, 3, 1, 0).reshape(_D, _C, _N)
    g = _gather_call(labels.reshape(1, _N).astype(jnp.int32), x_t)
    gp = jnp.pad(g, ((0, 0), (0, _NP - _N)))
    sp = jnp.concatenate(
        [scores.astype(_F32), jnp.full((_NP - _N,), -1.0, _F32)])
    out_t = _nms_call(gp, sp.reshape(1, _NP), sp.reshape(_NP, 1))
    return out_t.reshape(_M, _M, _NP).transpose(2, 0, 1)[:_N]
